# Initial kernel scaffold; baseline (speedup 1.0000x reference)
import jax
import jax.numpy as jnp
from jax.experimental import pallas as pl


def kernel(inputs, weights, W, b):
    raise NotImplementedError("write your pallas kernel here")


# speedup vs baseline: 1.0912x; 1.0912x over previous
import functools

import jax
import jax.numpy as jnp
from jax import lax
from jax.experimental import pallas as pl
from jax.experimental.pallas import tpu as pltpu
from jax.experimental.pallas import tpu_sc as plsc

_GATHER_WINDOW = 128
_PROJ_BLOCK = 8192


def _sc_gather(table, idx_flat):
    n = idx_flat.shape[0]
    _, d = table.shape
    idx2 = idx_flat.reshape(1, n)
    mesh = plsc.VectorSubcoreMesh(core_axis_name="c", subcore_axis_name="s")

    @functools.partial(
        pl.kernel,
        out_type=jax.ShapeDtypeStruct((n, d), table.dtype),
        mesh=mesh,
        compiler_params=pltpu.CompilerParams(use_tc_tiling_on_sc=False),
    )
    def gather_kernel(tab_hbm, i_hbm, o_hbm):
        def body(i_vmem, o_vmem):
            pltpu.sync_copy(tab_hbm.at[i_vmem.at[0]], o_vmem)

        pltpu.emit_pipeline(
            body,
            grid=(n // _GATHER_WINDOW,),
            in_specs=[pl.BlockSpec((1, _GATHER_WINDOW), lambda i: (0, i))],
            out_specs=[pl.BlockSpec((_GATHER_WINDOW, d), lambda i: (i, 0))],
            core_axis_name=("c", "s"),
            dimension_semantics=(pltpu.PARALLEL,),
        )(i_hbm, o_hbm)

    return gather_kernel(table, idx2)


def _tc_project(rows, w_t, b_tile):
    n, d = rows.shape
    p = w_t.shape[1]

    def body(x_ref, w_ref, b_ref, o_ref):
        acc = lax.dot_general(
            x_ref[...],
            w_ref[...],
            (((1,), (0,)), ((), ())),
            preferred_element_type=jnp.float32,
            precision=lax.Precision.HIGHEST,
        )
        o_ref[...] = acc + b_ref[0:1, :]

    return pl.pallas_call(
        body,
        grid=(n // _PROJ_BLOCK,),
        in_specs=[
            pl.BlockSpec((_PROJ_BLOCK, d), lambda i: (i, 0)),
            pl.BlockSpec((d, p), lambda i: (0, 0)),
            pl.BlockSpec((8, p), lambda i: (0, 0)),
        ],
        out_specs=pl.BlockSpec((_PROJ_BLOCK, p), lambda i: (i, 0)),
        out_shape=jax.ShapeDtypeStruct((n, p), jnp.float32),
    )(rows, w_t, b_tile)


def kernel(inputs, weights, W, b):
    batch, hist = inputs.shape
    proj = W.shape[0]
    n = batch * hist
    idx = inputs.reshape(n)
    emb = _sc_gather(weights, idx)
    w_t = W.T
    b_tile = jnp.broadcast_to(b[None, :], (8, proj))
    out = _tc_project(emb, w_t, b_tile)
    return out.reshape(batch, hist, proj)


# speedup vs baseline: 1.4684x; 1.3456x over previous
import functools

import jax
import jax.numpy as jnp
from jax import lax
from jax.experimental import pallas as pl
from jax.experimental.pallas import tpu as pltpu
from jax.experimental.pallas import tpu_sc as plsc

_GATHER_WINDOW = 128
_PROJ_BLOCK = 4096


def _sc_gather_packed(table, idx_flat):
    n = idx_flat.shape[0]
    _, d = table.shape
    idx2 = idx_flat.reshape(1, n)
    mesh = plsc.VectorSubcoreMesh(core_axis_name="c", subcore_axis_name="s")

    @functools.partial(
        pl.kernel,
        out_type=jax.ShapeDtypeStruct((n, d), table.dtype),
        mesh=mesh,
        compiler_params=pltpu.CompilerParams(use_tc_tiling_on_sc=False),
    )
    def gather_kernel(tab_hbm, i_hbm, o_hbm):
        def body(i_vmem, o_vmem):
            pltpu.sync_copy(tab_hbm.at[i_vmem.at[0]], o_vmem)

        pltpu.emit_pipeline(
            body,
            grid=(n // _GATHER_WINDOW,),
            in_specs=[pl.BlockSpec((1, _GATHER_WINDOW), lambda i: (0, i))],
            out_specs=[pl.BlockSpec((_GATHER_WINDOW, d), lambda i: (i, 0))],
            core_axis_name=("c", "s"),
            dimension_semantics=(pltpu.PARALLEL,),
        )(i_hbm, o_hbm)

    return gather_kernel(table, idx2)


def _tc_project_packed(rows, w_diag, b_tile):
    n2, dd = rows.shape

    def body(x_ref, w_ref, b_ref, o_ref):
        acc = lax.dot_general(
            x_ref[...],
            w_ref[...],
            (((1,), (0,)), ((), ())),
            preferred_element_type=jnp.float32,
            precision=lax.Precision.HIGHEST,
        )
        o_ref[...] = acc + b_ref[0:1, :]

    return pl.pallas_call(
        body,
        grid=(n2 // _PROJ_BLOCK,),
        in_specs=[
            pl.BlockSpec((_PROJ_BLOCK, dd), lambda i: (i, 0)),
            pl.BlockSpec((dd, dd), lambda i: (0, 0)),
            pl.BlockSpec((8, dd), lambda i: (0, 0)),
        ],
        out_specs=pl.BlockSpec((_PROJ_BLOCK, dd), lambda i: (i, 0)),
        out_shape=jax.ShapeDtypeStruct((n2, dd), jnp.float32),
    )(rows, w_diag, b_tile)


def kernel(inputs, weights, W, b):
    batch, hist = inputs.shape
    embed = weights.shape[1]
    proj = W.shape[0]
    n = batch * hist
    idx = inputs.reshape(n)
    emb = _sc_gather_packed(weights, idx)
    emb_packed = emb.reshape(n // 2, 2 * embed)
    w_t = W.T.astype(jnp.float32)
    w_diag = (
        jnp.zeros((2 * embed, 2 * proj), jnp.float32)
        .at[:embed, :proj].set(w_t)
        .at[embed:, proj:].set(w_t)
    )
    b2 = jnp.concatenate([b, b])
    b_tile = jnp.broadcast_to(b2[None, :], (8, 2 * proj))
    out_packed = _tc_project_packed(emb_packed, w_diag, b_tile)
    return out_packed.reshape(batch, hist, proj)
